# baseline (speedup 1.0000x reference)
import functools

import jax
import jax.numpy as jnp
from jax import lax
from jax.experimental import pallas as pl
from jax.experimental.pallas import tpu as pltpu
from jax.experimental.pallas import tpu_sc as plsc

B = 16384
D = 32
NC, NS = 2, 16
NW = NC * NS
BPW = B // NW
CH = 256
NCHUNK = BPW // CH

_mesh = plsc.VectorSubcoreMesh(
    core_axis_name="c", subcore_axis_name="s", num_cores=NC, num_subcores=NS
)


@functools.partial(
    pl.kernel,
    out_type=tuple(
        jax.ShapeDtypeStruct((B * D,), jnp.float32) for _ in range(7)),
    mesh=_mesh,
    scratch_types=[
        pltpu.VMEM((7, BPW), jnp.int32),
        pltpu.VMEM((7, CH, D), jnp.float32),
        pltpu.VMEM((7, CH * D), jnp.float32),
        pltpu.SemaphoreType.DMA,
        pltpu.SemaphoreType.DMA,
        pltpu.SemaphoreType.DMA,
    ],
    compiler_params=pltpu.CompilerParams(use_tc_tiling_on_sc=False),
)
def _gather7(u, o, f0, f1, r, d, t, Wu, Wo, Wf, Wr, Wd, Wh,
             o0, o1, o2, o3, o4, o5, o6,
             idx_v, rows_v, lin_v, sem_i, sem_g, sem_o):
    wid = lax.axis_index("s") * NC + lax.axis_index("c")
    base = wid * BPW
    idx_hbm = (u, o, f0, f1, r, d, t)
    tables = (Wu, Wo, Wf, Wf, Wr, Wd, Wh)
    outs = (o0, o1, o2, o3, o4, o5, o6)
    icps = [
        pltpu.async_copy(idx_hbm[i].at[pl.ds(base, BPW)], idx_v.at[i], sem_i)
        for i in range(7)
    ]
    for c in icps:
        c.wait()
    for h in range(NCHUNK):
        gcps = [
            pltpu.async_copy(tables[i].at[idx_v.at[i, pl.ds(h * CH, CH)]],
                             rows_v.at[i], sem_g)
            for i in range(7)
        ]
        ocps = []
        for i in range(7):
            gcps[i].wait()

            def body(rr, carry, i=i):
                for j in range(2):
                    lin_v[i, pl.ds(rr * D + j * 16, 16)] = (
                        rows_v[i, rr, pl.ds(j * 16, 16)])
                return carry

            lax.fori_loop(0, CH, body, 0)
            ocps.append(
                pltpu.async_copy(
                    lin_v.at[i],
                    outs[i].at[pl.ds((base + h * CH) * D, CH * D)], sem_o))
        for c in ocps:
            c.wait()


def kernel(user_id, organization, interested_fields_0, interested_fields_1,
           role, date, time, W_user, W_org, W_field, W_role, W_day, W_hour):
    es = _gather7(user_id, organization, interested_fields_0,
                  interested_fields_1, role, date, time,
                  W_user, W_org, W_field, W_role, W_day, W_hour)
    return jnp.concatenate([e.reshape(B, D) for e in es], axis=1)


# speedup vs baseline: 1.1372x; 1.1372x over previous
import functools

import jax
import jax.numpy as jnp
from jax import lax
from jax.experimental import pallas as pl
from jax.experimental.pallas import tpu as pltpu
from jax.experimental.pallas import tpu_sc as plsc

B = 16384
D = 32
NC, NS = 2, 16
NW = NC * NS
BPW = B // NW
HCH = BPW // 2

FIELD_V, ROLE_V, DAY_V, HOUR_V = 1000, 1000, 32, 24

_mesh = plsc.VectorSubcoreMesh(
    core_axis_name="c", subcore_axis_name="s", num_cores=NC, num_subcores=NS
)


@functools.partial(
    pl.kernel,
    out_type=jax.ShapeDtypeStruct((B, 7 * D), jnp.float32),
    mesh=_mesh,
    scratch_types=[
        pltpu.VMEM((7, BPW), jnp.int32),
        pltpu.VMEM((BPW, D), jnp.float32),
        pltpu.VMEM((BPW, D), jnp.float32),
        pltpu.VMEM((FIELD_V * D,), jnp.float32),
        pltpu.VMEM((ROLE_V * D,), jnp.float32),
        pltpu.VMEM((DAY_V * D,), jnp.float32),
        pltpu.VMEM((HOUR_V * D,), jnp.float32),
        pltpu.VMEM((HCH, D), jnp.float32),
        pltpu.VMEM((HCH, D), jnp.float32),
        pltpu.SemaphoreType.DMA,
        pltpu.SemaphoreType.DMA,
        pltpu.SemaphoreType.DMA,
        pltpu.SemaphoreType.DMA,
    ],
    compiler_params=pltpu.CompilerParams(use_tc_tiling_on_sc=False),
)
def _usermodel(u, o, f0, f1, r, d, t, Wu, Wo, wf1, wr1, wd1, wh1,
               out, idx_v, rows_u, rows_o, wf_v, wr_v, wd_v, wh_v,
               ext_a, ext_b, sem_i, sem_g, sem_t, sem_o):
    wid = lax.axis_index("s") * NC + lax.axis_index("c")
    base = wid * BPW
    idx_hbm = (u, o, f0, f1, r, d, t)

    icps = [
        pltpu.async_copy(idx_hbm[i].at[pl.ds(base, BPW)], idx_v.at[i], sem_i)
        for i in range(7)
    ]
    tcps = [
        pltpu.async_copy(src, dst, sem_t)
        for src, dst in ((wf1, wf_v), (wr1, wr_v), (wd1, wd_v), (wh1, wh_v))
    ]
    for c in icps:
        c.wait()
    gu = pltpu.async_copy(Wu.at[idx_v.at[0]], rows_u, sem_g)
    go = pltpu.async_copy(Wo.at[idx_v.at[1]], rows_o, sem_g)
    for c in tcps:
        c.wait()

    smalls = ((2, wf_v), (3, wf_v), (4, wr_v), (5, wd_v), (6, wh_v))
    prev = [None, None]
    nslot = 0
    for si, (feat, tab) in enumerate(smalls):
        for half in range(2):
            slot = nslot % 2
            nslot += 1
            buf = ext_a if slot == 0 else ext_b
            if prev[slot] is not None:
                prev[slot].wait()

            def body(g, carry, feat=feat, tab=tab, half=half, buf=buf):
                idx16 = idx_v[feat, pl.ds(half * HCH + g * 16, 16)] * D
                for k in range(16):
                    s = idx16[k]
                    r = g * 16 + k
                    buf[r, pl.ds(0, 16)] = tab[pl.ds(s, 16)]
                    buf[r, pl.ds(16, 16)] = tab[pl.ds(s + 16, 16)]
                return carry

            lax.fori_loop(0, HCH // 16, body, 0)
            prev[slot] = pltpu.async_copy(
                buf,
                out.at[pl.ds(base + half * HCH, HCH),
                       pl.ds(feat * D, D)], sem_o)

    gu.wait()
    cu = pltpu.async_copy(rows_u, out.at[pl.ds(base, BPW), pl.ds(0, D)],
                          sem_o)
    go.wait()
    co = pltpu.async_copy(rows_o, out.at[pl.ds(base, BPW), pl.ds(D, D)],
                          sem_o)
    for c in (prev[0], prev[1], cu, co):
        if c is not None:
            c.wait()


def kernel(user_id, organization, interested_fields_0, interested_fields_1,
           role, date, time, W_user, W_org, W_field, W_role, W_day, W_hour):
    return _usermodel(
        user_id, organization, interested_fields_0, interested_fields_1,
        role, date, time, W_user, W_org,
        W_field.reshape(-1), W_role.reshape(-1),
        W_day.reshape(-1), W_hour.reshape(-1))
